# baseline (speedup 1.0000x reference)
import functools

import jax
import jax.numpy as jnp
from jax import lax
from jax.experimental import pallas as pl
from jax.experimental.pallas import tpu as pltpu
from jax.experimental.pallas import tpu_sc as plsc

VOCAB = 1000000
PCA = 16
HIDDEN = 128
B = 16384

_info = plsc.get_sparse_core_info()
_NC, _NS = _info.num_cores, _info.num_subcores
NW = _NC * _NS
BH = B // 2
BPW = BH // NW
_W = 128
_NBUF = 16


def _gather_body(x_hbm, c_hbm, h_hbm, outc_hbm, outh_hbm,
                 idx_v, blkc_v, blkh_v, kvc_v, kvh_v, sem):
    wid = lax.axis_index("s") * _NC + lax.axis_index("c")
    base = wid * BPW
    pltpu.sync_copy(x_hbm.at[pl.ds(base, BPW)], idx_v)
    lanes = lax.iota(jnp.int32, 16)

    def group(g, carry):
        xvec = idx_v[pl.ds(g * _NBUF, _NBUF)]
        xs, copies = [], []
        for j in range(_NBUF):
            xj = xvec[j]
            off = pl.multiple_of(jnp.bitwise_and(xj, -_W), _W)
            xs.append(xj)
            copies.append(pltpu.async_copy(
                c_hbm.at[:, pl.ds(off, _W)], blkc_v.at[j], sem))
            copies.append(pltpu.async_copy(
                h_hbm.at[:, pl.ds(off, _W)], blkh_v.at[j], sem))
        for cp in copies:
            cp.wait()
        for j in range(_NBUF):
            i = g * _NBUF + j
            iv = jnp.full((16,), i, jnp.int32)
            jv = jnp.full((16,), j, jnp.int32)
            cv = jnp.full((16,), jnp.bitwise_and(xs[j], _W - 1), jnp.int32)
            plsc.store_scatter(
                kvc_v, [lanes, iv], plsc.load_gather(blkc_v, [jv, lanes, cv]))
            plsc.store_scatter(
                kvh_v, [lanes, iv], plsc.load_gather(blkh_v, [jv, lanes, cv]))
        return carry

    lax.fori_loop(0, BPW // _NBUF, group, 0)
    pltpu.sync_copy(kvc_v, outc_hbm.at[:, pl.ds(base, BPW)])
    pltpu.sync_copy(kvh_v, outh_hbm.at[:, pl.ds(base, BPW)])


_sc_gather = functools.partial(
    pl.kernel,
    mesh=plsc.VectorSubcoreMesh(core_axis_name="c", subcore_axis_name="s"),
    out_type=[jax.ShapeDtypeStruct((PCA, BH), jnp.float32),
              jax.ShapeDtypeStruct((PCA, BH), jnp.float32)],
    scratch_types=[
        pltpu.VMEM((BPW,), jnp.int32),
        pltpu.VMEM((_NBUF, PCA, _W), jnp.float32),
        pltpu.VMEM((_NBUF, PCA, _W), jnp.float32),
        pltpu.VMEM((PCA, BPW), jnp.float32),
        pltpu.VMEM((PCA, BPW), jnp.float32),
        pltpu.SemaphoreType.DMA,
    ],
    compiler_params=pltpu.CompilerParams(needs_layout_passes=False),
)(_gather_body)


_BB = 2048


def _proj_body(ec_ref, eh_ref, wc_ref, wh_ref, bc_ref, bh_ref,
               oc_ref, oh_ref):
    dn = (((0,), (0,)), ((), ()))
    oc_ref[...] = (
        lax.dot_general(ec_ref[...], wc_ref[...], dn,
                        preferred_element_type=jnp.float32)
        + bc_ref[...])
    oh_ref[...] = (
        lax.dot_general(eh_ref[...], wh_ref[...], dn,
                        preferred_element_type=jnp.float32)
        + bh_ref[...])


def _project(emb_c, emb_h, Wct, Wht, bc2, bh2):
    grid = BH // _BB
    return pl.pallas_call(
        _proj_body,
        grid=(grid,),
        in_specs=[
            pl.BlockSpec((PCA, _BB), lambda i: (0, i)),
            pl.BlockSpec((PCA, _BB), lambda i: (0, i)),
            pl.BlockSpec((PCA, HIDDEN), lambda i: (0, 0)),
            pl.BlockSpec((PCA, HIDDEN), lambda i: (0, 0)),
            pl.BlockSpec((1, HIDDEN), lambda i: (0, 0)),
            pl.BlockSpec((1, HIDDEN), lambda i: (0, 0)),
        ],
        out_specs=[
            pl.BlockSpec((_BB, HIDDEN), lambda i: (i, 0)),
            pl.BlockSpec((_BB, HIDDEN), lambda i: (i, 0)),
        ],
        out_shape=[
            jax.ShapeDtypeStruct((BH, HIDDEN), jnp.float32),
            jax.ShapeDtypeStruct((BH, HIDDEN), jnp.float32),
        ],
    )(emb_c, emb_h, Wct, Wht, bc2, bh2)


def kernel(x, c_table, h_table, Wc, bc, Wh, bh):
    xi = x.astype(jnp.int32)
    ctT, htT = c_table.T, h_table.T
    wct, wht = Wc.T, Wh.T
    bc2, bh2 = bc.reshape(1, HIDDEN), bh.reshape(1, HIDDEN)
    halves = []
    for h in range(2):
        ec_kv, eh_kv = _sc_gather(xi[h * BH:(h + 1) * BH], ctT, htT)
        halves.append(_project(ec_kv, eh_kv, wct, wht, bc2, bh2))
    oc = jnp.concatenate([halves[0][0], halves[1][0]], axis=0)
    oh = jnp.concatenate([halves[0][1], halves[1][1]], axis=0)
    return (oc.reshape(1, B, HIDDEN), oh.reshape(1, B, HIDDEN))


# speedup vs baseline: 1.1074x; 1.1074x over previous
import functools

import jax
import jax.numpy as jnp
from jax import lax
from jax.experimental import pallas as pl
from jax.experimental.pallas import tpu as pltpu
from jax.experimental.pallas import tpu_sc as plsc

VOCAB = 1000000
PCA = 16
HIDDEN = 128
B = 16384

_info = plsc.get_sparse_core_info()
_NC, _NS = _info.num_cores, _info.num_subcores
NW = _NC * _NS
BPW = B // NW
_W = 128
_NBUF = 16


def _gather_body(x_hbm, c_hbm, h_hbm, outc_hbm, outh_hbm,
                 idx_v, blkc_v, blkh_v, kvc_v, kvh_v, sem):
    wid = lax.axis_index("s") * _NC + lax.axis_index("c")
    base = wid * BPW
    pltpu.sync_copy(x_hbm.at[pl.ds(base, BPW)], idx_v)
    lanes = lax.iota(jnp.int32, 16)

    def group(g, carry):
        xvec = idx_v[pl.ds(g * _NBUF, _NBUF)]
        xs, copies = [], []
        for j in range(_NBUF):
            xj = xvec[j]
            off = pl.multiple_of(jnp.bitwise_and(xj, -_W), _W)
            xs.append(xj)
            copies.append(pltpu.async_copy(
                c_hbm.at[:, pl.ds(off, _W)], blkc_v.at[j], sem))
            copies.append(pltpu.async_copy(
                h_hbm.at[:, pl.ds(off, _W)], blkh_v.at[j], sem))
        for cp in copies:
            cp.wait()
        for j in range(_NBUF):
            i = g * _NBUF + j
            iv = jnp.full((16,), i, jnp.int32)
            jv = jnp.full((16,), j, jnp.int32)
            cv = jnp.full((16,), jnp.bitwise_and(xs[j], _W - 1), jnp.int32)
            plsc.store_scatter(
                kvc_v, [lanes, iv], plsc.load_gather(blkc_v, [jv, lanes, cv]))
            plsc.store_scatter(
                kvh_v, [lanes, iv], plsc.load_gather(blkh_v, [jv, lanes, cv]))
        return carry

    lax.fori_loop(0, BPW // _NBUF, group, 0)
    pltpu.sync_copy(kvc_v, outc_hbm.at[:, pl.ds(base, BPW)])
    pltpu.sync_copy(kvh_v, outh_hbm.at[:, pl.ds(base, BPW)])


_sc_gather = functools.partial(
    pl.kernel,
    mesh=plsc.VectorSubcoreMesh(core_axis_name="c", subcore_axis_name="s"),
    out_type=[jax.ShapeDtypeStruct((PCA, B), jnp.float32),
              jax.ShapeDtypeStruct((PCA, B), jnp.float32)],
    scratch_types=[
        pltpu.VMEM((BPW,), jnp.int32),
        pltpu.VMEM((_NBUF, PCA, _W), jnp.float32),
        pltpu.VMEM((_NBUF, PCA, _W), jnp.float32),
        pltpu.VMEM((PCA, BPW), jnp.float32),
        pltpu.VMEM((PCA, BPW), jnp.float32),
        pltpu.SemaphoreType.DMA,
    ],
    compiler_params=pltpu.CompilerParams(needs_layout_passes=False),
)(_gather_body)


_BB = 2048


def _proj_body(ec_ref, eh_ref, wc_ref, wh_ref, bc_ref, bh_ref,
               oc_ref, oh_ref):
    dn = (((0,), (0,)), ((), ()))
    oc_ref[...] = (
        lax.dot_general(ec_ref[...], wc_ref[...], dn,
                        preferred_element_type=jnp.float32)
        + bc_ref[...])
    oh_ref[...] = (
        lax.dot_general(eh_ref[...], wh_ref[...], dn,
                        preferred_element_type=jnp.float32)
        + bh_ref[...])


def _project(emb_c, emb_h, Wct, Wht, bc2, bh2):
    grid = B // _BB
    return pl.pallas_call(
        _proj_body,
        grid=(grid,),
        in_specs=[
            pl.BlockSpec((PCA, _BB), lambda i: (0, i)),
            pl.BlockSpec((PCA, _BB), lambda i: (0, i)),
            pl.BlockSpec((PCA, HIDDEN), lambda i: (0, 0)),
            pl.BlockSpec((PCA, HIDDEN), lambda i: (0, 0)),
            pl.BlockSpec((1, HIDDEN), lambda i: (0, 0)),
            pl.BlockSpec((1, HIDDEN), lambda i: (0, 0)),
        ],
        out_specs=[
            pl.BlockSpec((_BB, HIDDEN), lambda i: (i, 0)),
            pl.BlockSpec((_BB, HIDDEN), lambda i: (i, 0)),
        ],
        out_shape=[
            jax.ShapeDtypeStruct((B, HIDDEN), jnp.float32),
            jax.ShapeDtypeStruct((B, HIDDEN), jnp.float32),
        ],
    )(emb_c, emb_h, Wct, Wht, bc2, bh2)


def kernel(x, c_table, h_table, Wc, bc, Wh, bh):
    xi = x.astype(jnp.int32)
    ec_kv, eh_kv = _sc_gather(xi, c_table.T, h_table.T)
    oc, oh = _project(ec_kv, eh_kv, Wc.T, Wh.T,
                      bc.reshape(1, HIDDEN), bh.reshape(1, HIDDEN))
    return (oc.reshape(1, B, HIDDEN), oh.reshape(1, B, HIDDEN))


# speedup vs baseline: 1.1172x; 1.0088x over previous
import functools

import jax
import jax.numpy as jnp
from jax import lax
from jax.experimental import pallas as pl
from jax.experimental.pallas import tpu as pltpu
from jax.experimental.pallas import tpu_sc as plsc

VOCAB = 1000000
PCA = 16
HIDDEN = 128
B = 16384

_info = plsc.get_sparse_core_info()
_NC, _NS = _info.num_cores, _info.num_subcores
NW = _NC * _NS
BPW = B // NW
_W = 128
_NBUF = 16


def _gather_body(x_hbm, c_hbm, h_hbm, outc_hbm, outh_hbm,
                 idx_v, blkc_v, blkh_v, kvc_v, kvh_v, sem):
    wid = lax.axis_index("s") * _NC + lax.axis_index("c")
    base = wid * BPW
    pltpu.sync_copy(x_hbm.at[pl.ds(base, BPW)], idx_v)
    lanes = lax.iota(jnp.int32, 16)

    def group(g, carry):
        xvec = idx_v[pl.ds(g * _NBUF, _NBUF)]
        xs, copies = [], []
        for j in range(_NBUF):
            xj = xvec[j]
            off = pl.multiple_of(jnp.bitwise_and(xj, -_W), _W)
            xs.append(xj)
            copies.append(pltpu.async_copy(
                c_hbm.at[:, pl.ds(off, _W)], blkc_v.at[j], sem))
            copies.append(pltpu.async_copy(
                h_hbm.at[:, pl.ds(off, _W)], blkh_v.at[j], sem))
        for cp in copies:
            cp.wait()
        for j in range(_NBUF):
            i = g * _NBUF + j
            iv = jnp.full((16,), i, jnp.int32)
            jv = jnp.full((16,), j, jnp.int32)
            cv = jnp.full((16,), jnp.bitwise_and(xs[j], _W - 1), jnp.int32)
            plsc.store_scatter(
                kvc_v, [lanes, iv], plsc.load_gather(blkc_v, [jv, lanes, cv]))
            plsc.store_scatter(
                kvh_v, [lanes, iv], plsc.load_gather(blkh_v, [jv, lanes, cv]))
        return carry

    lax.fori_loop(0, BPW // _NBUF, group, 0)
    pltpu.sync_copy(kvc_v, outc_hbm.at[:, pl.ds(base, BPW)])
    pltpu.sync_copy(kvh_v, outh_hbm.at[:, pl.ds(base, BPW)])


_sc_gather = functools.partial(
    pl.kernel,
    mesh=plsc.VectorSubcoreMesh(core_axis_name="c", subcore_axis_name="s"),
    out_type=[jax.ShapeDtypeStruct((PCA, B), jnp.float32),
              jax.ShapeDtypeStruct((PCA, B), jnp.float32)],
    scratch_types=[
        pltpu.VMEM((BPW,), jnp.int32),
        pltpu.VMEM((_NBUF, PCA, _W), jnp.float32),
        pltpu.VMEM((_NBUF, PCA, _W), jnp.float32),
        pltpu.VMEM((PCA, BPW), jnp.float32),
        pltpu.VMEM((PCA, BPW), jnp.float32),
        pltpu.SemaphoreType.DMA,
    ],
    compiler_params=pltpu.CompilerParams(needs_layout_passes=False),
)(_gather_body)


_BB = 8192


def _proj_body(ec_ref, eh_ref, wc_ref, wh_ref, bc_ref, bh_ref,
               oc_ref, oh_ref):
    dn = (((0,), (0,)), ((), ()))
    oc_ref[...] = (
        lax.dot_general(ec_ref[...], wc_ref[...], dn,
                        preferred_element_type=jnp.float32)
        + bc_ref[...])
    oh_ref[...] = (
        lax.dot_general(eh_ref[...], wh_ref[...], dn,
                        preferred_element_type=jnp.float32)
        + bh_ref[...])


def _project(emb_c, emb_h, Wct, Wht, bc2, bh2):
    grid = B // _BB
    return pl.pallas_call(
        _proj_body,
        grid=(grid,),
        in_specs=[
            pl.BlockSpec((PCA, _BB), lambda i: (0, i)),
            pl.BlockSpec((PCA, _BB), lambda i: (0, i)),
            pl.BlockSpec((PCA, HIDDEN), lambda i: (0, 0)),
            pl.BlockSpec((PCA, HIDDEN), lambda i: (0, 0)),
            pl.BlockSpec((1, HIDDEN), lambda i: (0, 0)),
            pl.BlockSpec((1, HIDDEN), lambda i: (0, 0)),
        ],
        out_specs=[
            pl.BlockSpec((_BB, HIDDEN), lambda i: (i, 0)),
            pl.BlockSpec((_BB, HIDDEN), lambda i: (i, 0)),
        ],
        out_shape=[
            jax.ShapeDtypeStruct((B, HIDDEN), jnp.float32),
            jax.ShapeDtypeStruct((B, HIDDEN), jnp.float32),
        ],
    )(emb_c, emb_h, Wct, Wht, bc2, bh2)


def kernel(x, c_table, h_table, Wc, bc, Wh, bh):
    xi = x.astype(jnp.int32)
    ec_kv, eh_kv = _sc_gather(xi, c_table.T, h_table.T)
    oc, oh = _project(ec_kv, eh_kv, Wc.T, Wh.T,
                      bc.reshape(1, HIDDEN), bh.reshape(1, HIDDEN))
    return (oc.reshape(1, B, HIDDEN), oh.reshape(1, B, HIDDEN))
